# baseline (speedup 1.0000x reference)
import jax
import jax.numpy as jnp
from jax.experimental import pallas as pl

_SEQ_BLOCK = 256


def _add_kernel(x_ref, p_ref, o_ref):
    o_ref[...] = x_ref[...] + p_ref[...]


def kernel(inputs, pos_table):
    batch, seq, dim = inputs.shape
    nblk = seq // _SEQ_BLOCK
    return pl.pallas_call(
        _add_kernel,
        grid=(nblk,),
        in_specs=[
            pl.BlockSpec((batch, _SEQ_BLOCK, dim), lambda i: (0, i, 0)),
            pl.BlockSpec((_SEQ_BLOCK, dim), lambda i: (i, 0)),
        ],
        out_specs=pl.BlockSpec((batch, _SEQ_BLOCK, dim), lambda i: (0, i, 0)),
        out_shape=jax.ShapeDtypeStruct((batch, seq, dim), inputs.dtype),
    )(inputs, pos_table)


# speedup vs baseline: 1.0282x; 1.0282x over previous
import jax
import jax.numpy as jnp
from jax.experimental import pallas as pl

_SEQ_BLOCK = 1024


def _add_kernel(x_ref, p_ref, o_ref):
    o_ref[...] = x_ref[...] + p_ref[...]


def kernel(inputs, pos_table):
    batch, seq, dim = inputs.shape
    nblk = seq // _SEQ_BLOCK
    return pl.pallas_call(
        _add_kernel,
        grid=(nblk,),
        in_specs=[
            pl.BlockSpec((batch, _SEQ_BLOCK, dim), lambda i: (0, i, 0)),
            pl.BlockSpec((_SEQ_BLOCK, dim), lambda i: (i, 0)),
        ],
        out_specs=pl.BlockSpec((batch, _SEQ_BLOCK, dim), lambda i: (0, i, 0)),
        out_shape=jax.ShapeDtypeStruct((batch, seq, dim), inputs.dtype),
    )(inputs, pos_table)
